# Initial kernel scaffold; baseline (speedup 1.0000x reference)
import jax
import jax.numpy as jnp
from jax.experimental import pallas as pl


def kernel(x, indices_to_select):
    raise NotImplementedError("write your pallas kernel here")


# speedup vs baseline: 3.2190x; 3.2190x over previous
import functools

import jax
import jax.numpy as jnp
from jax import lax
from jax.experimental import pallas as pl
from jax.experimental.pallas import tpu as pltpu
from jax.experimental.pallas import tpu_sc as plsc

NC = 2
NS = 16
NW = NC * NS

B = 4096
CH = 128
NBUF = 4
NCH = B // CH


def _gather_kernel(x_hbm, idx_hbm, out_hbm, idx_v, *scratch):
    bufs = scratch[:NBUF]
    gsems = scratch[NBUF:2 * NBUF]
    ssems = scratch[2 * NBUF:]

    wid = lax.axis_index("s") * NC + lax.axis_index("c")
    table = x_hbm.at[wid]
    outw = out_hbm.at[wid]

    pltpu.sync_copy(idx_hbm, idx_v)

    gather_h = [None] * NCH
    store_h = [None] * NCH

    def start_gather(i):
        b = i % NBUF
        gather_h[i] = pltpu.async_copy(
            table.at[idx_v.at[pl.ds(i * CH, CH)]], bufs[b], gsems[b])

    for i in range(min(NBUF, NCH)):
        start_gather(i)
    for i in range(NCH):
        b = i % NBUF
        gather_h[i].wait()
        store_h[i] = pltpu.async_copy(
            bufs[b], outw.at[pl.ds(i * CH, CH)], ssems[b])
        j = i + NBUF
        if j < NCH:
            store_h[i].wait()
            start_gather(j)
    for i in range(max(0, NCH - NBUF), NCH):
        store_h[i].wait()


@jax.jit
def _direct_slice(x3, idx):
    scratch = (
        [pltpu.VMEM((B,), jnp.int32)]
        + [pltpu.VMEM((CH, 128), jnp.float32) for _ in range(NBUF)]
        + [pltpu.SemaphoreType.DMA for _ in range(2 * NBUF)]
    )
    run = pl.kernel(
        _gather_kernel,
        out_type=jax.ShapeDtypeStruct((NW, B, 128), jnp.float32),
        mesh=plsc.VectorSubcoreMesh(core_axis_name="c", subcore_axis_name="s"),
        scratch_types=scratch,
    )
    return run(x3, idx)


def kernel(x, indices_to_select):
    bb, hh, v, d = x.shape
    x3 = x.reshape(bb * hh, v, d)
    out3 = _direct_slice(x3, indices_to_select.astype(jnp.int32))
    return out3.reshape(bb, hh, indices_to_select.shape[0], d)
